# Initial kernel scaffold; baseline (speedup 1.0000x reference)
import jax
import jax.numpy as jnp
from jax.experimental import pallas as pl


def kernel(path, embed):
    raise NotImplementedError("write your pallas kernel here")


# speedup vs baseline: 2.1963x; 2.1963x over previous
import functools

import jax
import jax.numpy as jnp
from jax import lax
from jax.experimental import pallas as pl
from jax.experimental.pallas import tpu as pltpu
from jax.experimental.pallas import tpu_sc as plsc

_DIM = 512
_P = 16
_NW = 32
_NC = 2


def _body(idx_hbm, embed_hbm, *rest):
    outs = rest[:_P]
    idx_v, buf_a, buf_b, sem_a, sem_b = rest[_P:]
    n_rows = outs[0].shape[0]
    rows_per_w = n_rows // _NW
    nch = idx_v.shape[1]
    ch = idx_v.shape[2]

    wid = lax.axis_index("s") * _NC + lax.axis_index("c")
    base = wid * rows_per_w

    pltpu.sync_copy(idx_hbm.at[wid], idx_v)

    for s in range(_P):
        out = outs[s]

        @pl.loop(0, nch, step=2)
        def _(c):
            ga = pltpu.async_copy(embed_hbm.at[idx_v.at[s, c]], buf_a, sem_a)
            gb = pltpu.async_copy(embed_hbm.at[idx_v.at[s, c + 1]], buf_b, sem_b)
            ga.wait()
            pltpu.sync_copy(buf_a, out.at[pl.ds(base + c * ch, ch)])
            gb.wait()
            pltpu.sync_copy(buf_b, out.at[pl.ds(base + (c + 1) * ch, ch)])


def kernel(path, embed):
    b, s, p = path.shape
    n = b * s
    rows_per_w = n // _NW
    ch = 64
    nch = rows_per_w // ch

    idx = jnp.transpose(path.reshape(n, p)).reshape(p, _NW, nch, ch)
    idx = jnp.transpose(idx, (1, 0, 2, 3))

    mesh = plsc.VectorSubcoreMesh(core_axis_name="c", subcore_axis_name="s")
    run = pl.kernel(
        _body,
        out_type=[jax.ShapeDtypeStruct((n, _DIM), jnp.float32)] * _P,
        mesh=mesh,
        scratch_types=[
            pltpu.VMEM((p, nch, ch), jnp.int32),
            pltpu.VMEM((ch, _DIM), jnp.float32),
            pltpu.VMEM((ch, _DIM), jnp.float32),
            pltpu.SemaphoreType.DMA,
            pltpu.SemaphoreType.DMA,
        ],
    )
    outs = run(idx, embed)
    return tuple(o.reshape(b, s, _DIM) for o in outs)
